# speedup vs baseline: 1.3889x; 1.3889x over previous
import functools

import jax
import jax.numpy as jnp
from jax import lax
from jax.experimental import pallas as pl
from jax.experimental.pallas import tpu as pltpu
from jax.experimental.pallas import tpu_sc as plsc

N_NODES = 100000
EMB_DIM = 256
LANES = 16
NUM_WORKERS = 32
CHUNK = 80
NUM_CHUNKS = N_NODES // CHUNK
CHUNKS_PER_WORKER = -(-NUM_CHUNKS // NUM_WORKERS)


def _make_sc_kernel():
  mesh = plsc.VectorSubcoreMesh(core_axis_name="c", subcore_axis_name="s")

  @functools.partial(
      pl.kernel,
      mesh=mesh,
      out_type=jax.ShapeDtypeStruct((N_NODES, EMB_DIM), jnp.float32),
      scratch_types=[
          pltpu.VMEM((CHUNK,), jnp.int32),
          pltpu.VMEM((CHUNK, EMB_DIM), jnp.float32),
          pltpu.VMEM((CHUNK, EMB_DIM), jnp.float32),
          pltpu.SemaphoreType.DMA,
      ],
  )
  def k(x_hbm, deg_hbm, table_hbm, out_hbm, idx_v, emb_v, x_v, sem):
    wid = lax.axis_index("s") * 2 + lax.axis_index("c")

    def chunk_body(j, carry):
      g = wid + j * NUM_WORKERS

      @pl.when(g < NUM_CHUNKS)
      def _():
        base = g * CHUNK
        pltpu.sync_copy(deg_hbm.at[pl.ds(base, CHUNK)], idx_v)
        gather = pltpu.async_copy(table_hbm.at[idx_v], emb_v, sem)
        pltpu.sync_copy(x_hbm.at[pl.ds(base, CHUNK)], x_v)
        gather.wait()

        def node_body(i, c):
          for kk in range(EMB_DIM // LANES):
            sl = pl.ds(kk * LANES, LANES)
            x_v[i, sl] = x_v[i, sl] + emb_v[i, sl]
          return c

        lax.fori_loop(0, CHUNK, node_body, 0)
        pltpu.sync_copy(x_v, out_hbm.at[pl.ds(base, CHUNK)])

      return carry

    lax.fori_loop(0, CHUNKS_PER_WORKER, chunk_body, 0)

  return k


_sc_kernel = _make_sc_kernel()


@jax.jit
def kernel(x, deg, deg_emb_table):
  return _sc_kernel(x, deg, deg_emb_table)


# speedup vs baseline: 1.4625x; 1.0530x over previous
import functools

import jax
import jax.numpy as jnp
from jax import lax
from jax.experimental import pallas as pl
from jax.experimental.pallas import tpu as pltpu
from jax.experimental.pallas import tpu_sc as plsc

N_NODES = 100000
EMB_DIM = 256
LANES = 16
NUM_WORKERS = 32
CHUNK = 80
NUM_CHUNKS = N_NODES // CHUNK
CPW = NUM_CHUNKS // NUM_WORKERS
NUM_TAIL = NUM_CHUNKS - CPW * NUM_WORKERS
NSTEPS = CPW + 1


def _make_sc_kernel():
  mesh = plsc.VectorSubcoreMesh(core_axis_name="c", subcore_axis_name="s")

  @functools.partial(
      pl.kernel,
      mesh=mesh,
      out_type=jax.ShapeDtypeStruct((N_NODES, EMB_DIM), jnp.float32),
      scratch_types=[
          pltpu.VMEM((2, CHUNK), jnp.int32),
          pltpu.VMEM((2, CHUNK, EMB_DIM), jnp.float32),
          pltpu.VMEM((2, CHUNK, EMB_DIM), jnp.float32),
          pltpu.SemaphoreType.DMA,
          pltpu.SemaphoreType.DMA,
          pltpu.SemaphoreType.DMA,
          pltpu.SemaphoreType.DMA,
          pltpu.SemaphoreType.DMA,
          pltpu.SemaphoreType.DMA,
          pltpu.SemaphoreType.DMA,
          pltpu.SemaphoreType.DMA,
      ],
  )
  def k(x_hbm, deg_hbm, table_hbm, out_hbm, idx_v, emb_v, x_v,
        si0, si1, sg0, sg1, sx0, sx1, so0, so1):
    si = (si0, si1)
    sg = (sg0, sg1)
    sx = (sx0, sx1)
    so = (so0, so1)
    wid = lax.axis_index("s") * 2 + lax.axis_index("c")

    def active(t):
      return jnp.logical_and(t <= CPW, jnp.logical_or(t < CPW, wid < NUM_TAIL))

    def chunk_id(t):
      return jnp.where(t < CPW, wid * CPW + t, CPW * NUM_WORKERS + wid)

    def base(t):
      return chunk_id(t) * CHUNK

    def wait_idx(b):
      pltpu.make_async_copy(deg_hbm.at[pl.ds(0, CHUNK)], idx_v.at[b], si[b]).wait()

    def wait_gather(b):
      pltpu.make_async_copy(table_hbm.at[idx_v.at[b]], emb_v.at[b], sg[b]).wait()

    def wait_x(b):
      pltpu.make_async_copy(x_hbm.at[pl.ds(0, CHUNK)], x_v.at[b], sx[b]).wait()

    def wait_out(b):
      pltpu.make_async_copy(x_v.at[b], out_hbm.at[pl.ds(0, CHUNK)], so[b]).wait()

    pltpu.sync_copy(deg_hbm.at[pl.ds(base(0), CHUNK)], idx_v.at[0])
    pltpu.async_copy(table_hbm.at[idx_v.at[0]], emb_v.at[0], sg[0])
    pltpu.async_copy(x_hbm.at[pl.ds(base(0), CHUNK)], x_v.at[0], sx[0])
    pltpu.async_copy(deg_hbm.at[pl.ds(base(1), CHUNK)], idx_v.at[1], si[1])

    def step(j, a, o):
      @pl.when(j >= 1)
      def _():
        wait_out(o)

      @pl.when(active(j + 1))
      def _():
        wait_idx(o)
        pltpu.async_copy(table_hbm.at[idx_v.at[o]], emb_v.at[o], sg[o])
        pltpu.async_copy(x_hbm.at[pl.ds(base(j + 1), CHUNK)], x_v.at[o], sx[o])

      @pl.when(active(j))
      def _():
        wait_x(a)
        wait_gather(a)

        @pl.when(active(j + 2))
        def _():
          pltpu.async_copy(deg_hbm.at[pl.ds(base(j + 2), CHUNK)],
                           idx_v.at[a], si[a])

        xa = x_v.at[a]
        ea = emb_v.at[a]

        @plsc.parallel_loop(0, CHUNK, 1, unroll=2)
        def _(n):
          for kk in range(EMB_DIM // LANES):
            sl = pl.ds(kk * LANES, LANES)
            xa[n, sl] = xa[n, sl] + ea[n, sl]

        pltpu.async_copy(xa, out_hbm.at[pl.ds(base(j), CHUNK)], so[a])

    def pair_body(p, carry):
      step(2 * p, 0, 1)
      step(2 * p + 1, 1, 0)
      return carry

    lax.fori_loop(0, NSTEPS // 2, pair_body, 0)

    @pl.when(wid < NUM_TAIL)
    def _():
      wait_out(1)

  return k


_sc_kernel = _make_sc_kernel()


@jax.jit
def kernel(x, deg, deg_emb_table):
  return _sc_kernel(x, deg, deg_emb_table)


# speedup vs baseline: 2.7140x; 1.8557x over previous
import functools

import jax
import jax.numpy as jnp
from jax import lax
from jax.experimental import pallas as pl
from jax.experimental.pallas import tpu as pltpu
from jax.experimental.pallas import tpu_sc as plsc

N_NODES = 100000
EMB_DIM = 256
TABLE_ROWS = 150
LANES = 16
NUM_WORKERS = 32
CHUNK = 80
NUM_CHUNKS = N_NODES // CHUNK
CPW = NUM_CHUNKS // NUM_WORKERS
NUM_TAIL = NUM_CHUNKS - CPW * NUM_WORKERS
NSTEPS = CPW + 1


def _make_sc_kernel():
  mesh = plsc.VectorSubcoreMesh(core_axis_name="c", subcore_axis_name="s")

  @functools.partial(
      pl.kernel,
      mesh=mesh,
      compiler_params=pltpu.CompilerParams(needs_layout_passes=False),
      out_type=jax.ShapeDtypeStruct((N_NODES, EMB_DIM), jnp.float32),
      scratch_types=[
          pltpu.VMEM((TABLE_ROWS, EMB_DIM), jnp.float32),
          pltpu.VMEM((2, CHUNK), jnp.int32),
          pltpu.VMEM((2, CHUNK, EMB_DIM), jnp.float32),
          pltpu.SemaphoreType.DMA,
          pltpu.SemaphoreType.DMA,
          pltpu.SemaphoreType.DMA,
          pltpu.SemaphoreType.DMA,
          pltpu.SemaphoreType.DMA,
          pltpu.SemaphoreType.DMA,
      ],
  )
  def k(x_hbm, deg_hbm, table_hbm, out_hbm, table_v, idx_v, x_v,
        si0, si1, sx0, sx1, so0, so1):
    si = (si0, si1)
    sx = (sx0, sx1)
    so = (so0, so1)
    wid = lax.axis_index("s") * 2 + lax.axis_index("c")

    def active(t):
      return jnp.logical_and(t <= CPW, jnp.logical_or(t < CPW, wid < NUM_TAIL))

    def base(t):
      return jnp.where(t < CPW, wid * CPW + t, CPW * NUM_WORKERS + wid) * CHUNK

    def wait_idx(b):
      pltpu.make_async_copy(deg_hbm.at[pl.ds(0, CHUNK)], idx_v.at[b], si[b]).wait()

    def wait_x(b):
      pltpu.make_async_copy(x_hbm.at[pl.ds(0, CHUNK)], x_v.at[b], sx[b]).wait()

    def wait_out(b):
      pltpu.make_async_copy(x_v.at[b], out_hbm.at[pl.ds(0, CHUNK)], so[b]).wait()

    pltpu.async_copy(deg_hbm.at[pl.ds(base(0), CHUNK)], idx_v.at[0], si[0])
    pltpu.async_copy(x_hbm.at[pl.ds(base(0), CHUNK)], x_v.at[0], sx[0])
    pltpu.async_copy(deg_hbm.at[pl.ds(base(1), CHUNK)], idx_v.at[1], si[1])
    pltpu.sync_copy(table_hbm, table_v)

    def step(j, a, o):
      @pl.when(j >= 1)
      def _():
        wait_out(o)

      @pl.when(active(j + 1))
      def _():
        pltpu.async_copy(x_hbm.at[pl.ds(base(j + 1), CHUNK)], x_v.at[o], sx[o])

      @pl.when(active(j))
      def _():
        wait_x(a)
        wait_idx(a)

        xa = x_v.at[a]
        ia = idx_v.at[a]

        @plsc.parallel_loop(0, CHUNK, 1, unroll=2)
        def _(n):
          dsplat = plsc.load_gather(ia, [jnp.broadcast_to(n, (LANES,))])
          for kk in range(EMB_DIM // LANES):
            col = lax.iota(jnp.int32, LANES) + (kk * LANES)
            emb = plsc.load_gather(table_v, [dsplat, col])
            sl = pl.ds(kk * LANES, LANES)
            xa[n, sl] = xa[n, sl] + emb

        @pl.when(active(j + 2))
        def _():
          pltpu.async_copy(deg_hbm.at[pl.ds(base(j + 2), CHUNK)],
                           idx_v.at[a], si[a])

        pltpu.async_copy(xa, out_hbm.at[pl.ds(base(j), CHUNK)], so[a])

    def pair_body(p, carry):
      step(2 * p, 0, 1)
      step(2 * p + 1, 1, 0)
      return carry

    lax.fori_loop(0, NSTEPS // 2, pair_body, 0)

    @pl.when(wid < NUM_TAIL)
    def _():
      wait_out(1)

  return k


_sc_kernel = _make_sc_kernel()


@jax.jit
def kernel(x, deg, deg_emb_table):
  return _sc_kernel(x, deg, deg_emb_table)


# speedup vs baseline: 2.7426x; 1.0105x over previous
import functools

import jax
import jax.numpy as jnp
from jax import lax
from jax.experimental import pallas as pl
from jax.experimental.pallas import tpu as pltpu
from jax.experimental.pallas import tpu_sc as plsc

N_NODES = 100000
EMB_DIM = 256
TABLE_ROWS = 150
LANES = 16
NUM_WORKERS = 32
CHUNK = 80
NUM_CHUNKS = N_NODES // CHUNK
CPW = NUM_CHUNKS // NUM_WORKERS
NUM_TAIL = NUM_CHUNKS - CPW * NUM_WORKERS
NSTEPS = CPW + 1


def _make_sc_kernel():
  mesh = plsc.VectorSubcoreMesh(core_axis_name="c", subcore_axis_name="s")

  @functools.partial(
      pl.kernel,
      mesh=mesh,
      compiler_params=pltpu.CompilerParams(needs_layout_passes=False),
      out_type=jax.ShapeDtypeStruct((N_NODES, EMB_DIM), jnp.float32),
      scratch_types=[
          pltpu.VMEM((TABLE_ROWS, EMB_DIM), jnp.float32),
          pltpu.VMEM((2, CHUNK), jnp.int32),
          pltpu.VMEM((2, CHUNK, EMB_DIM), jnp.float32),
          pltpu.SemaphoreType.DMA,
          pltpu.SemaphoreType.DMA,
          pltpu.SemaphoreType.DMA,
          pltpu.SemaphoreType.DMA,
          pltpu.SemaphoreType.DMA,
          pltpu.SemaphoreType.DMA,
      ],
  )
  def k(x_hbm, deg_hbm, table_hbm, out_hbm, table_v, idx_v, x_v,
        si0, si1, sx0, sx1, so0, so1):
    si = (si0, si1)
    sx = (sx0, sx1)
    so = (so0, so1)
    wid = lax.axis_index("s") * 2 + lax.axis_index("c")

    def active(t):
      return jnp.logical_and(t <= CPW, jnp.logical_or(t < CPW, wid < NUM_TAIL))

    def base(t):
      return jnp.where(t < CPW, wid * CPW + t, CPW * NUM_WORKERS + wid) * CHUNK

    def wait_idx(b):
      pltpu.make_async_copy(deg_hbm.at[pl.ds(0, CHUNK)], idx_v.at[b], si[b]).wait()

    def wait_x(b):
      pltpu.make_async_copy(x_hbm.at[pl.ds(0, CHUNK)], x_v.at[b], sx[b]).wait()

    def wait_out(b):
      pltpu.make_async_copy(x_v.at[b], out_hbm.at[pl.ds(0, CHUNK)], so[b]).wait()

    pltpu.async_copy(deg_hbm.at[pl.ds(base(0), CHUNK)], idx_v.at[0], si[0])
    pltpu.async_copy(x_hbm.at[pl.ds(base(0), CHUNK)], x_v.at[0], sx[0])
    pltpu.async_copy(deg_hbm.at[pl.ds(base(1), CHUNK)], idx_v.at[1], si[1])
    pltpu.sync_copy(table_hbm, table_v)

    def step(j, a, o):
      @pl.when(j >= 1)
      def _():
        wait_out(o)

      @pl.when(active(j + 1))
      def _():
        pltpu.async_copy(x_hbm.at[pl.ds(base(j + 1), CHUNK)], x_v.at[o], sx[o])

      @pl.when(active(j))
      def _():
        wait_x(a)
        wait_idx(a)

        xa = x_v.at[a]
        ia = idx_v.at[a]

        @plsc.parallel_loop(0, CHUNK, 1, unroll=2)
        def _(n):
          nsplat = jnp.broadcast_to(n, (LANES,))
          dsplat = plsc.load_gather(ia, [nsplat])
          for kk in range(EMB_DIM // LANES):
            col = lax.iota(jnp.int32, LANES) + (kk * LANES)
            emb = plsc.load_gather(table_v, [dsplat, col])
            plsc.addupdate_scatter(xa, [nsplat, col], emb)

        @pl.when(active(j + 2))
        def _():
          pltpu.async_copy(deg_hbm.at[pl.ds(base(j + 2), CHUNK)],
                           idx_v.at[a], si[a])

        pltpu.async_copy(xa, out_hbm.at[pl.ds(base(j), CHUNK)], so[a])

    def pair_body(p, carry):
      step(2 * p, 0, 1)
      step(2 * p + 1, 1, 0)
      return carry

    lax.fori_loop(0, NSTEPS // 2, pair_body, 0)

    @pl.when(wid < NUM_TAIL)
    def _():
      wait_out(1)

  return k


_sc_kernel = _make_sc_kernel()


@jax.jit
def kernel(x, deg, deg_emb_table):
  return _sc_kernel(x, deg, deg_emb_table)
